# baseline (speedup 1.0000x reference)
import jax
import jax.numpy as jnp
from jax import lax
from jax.experimental import pallas as pl
from jax.experimental.pallas import tpu as pltpu, tpu_sc as plsc

VOCAB = 6
HIDDEN = 768
BATCH = 4
SEQ = 8192
ROWS = BATCH * SEQ
NC, NS = 2, 16
NW = NC * NS
ROWS_PER_W = ROWS // NW
C = 64
NCHUNK = ROWS_PER_W // C
L = 16
IN_STRIP = ROWS_PER_W * VOCAB


def _sc_pad_body(in_hbm, out_hbm, in_v, buf0, buf1, si, so0, so1):
    wid = lax.axis_index("s") * NC + lax.axis_index("c")
    row0 = wid * ROWS_PER_W
    b = row0 // SEQ
    rb0 = row0 - b * SEQ

    zero = jnp.zeros((L,), jnp.float32)
    mask6 = (lax.iota(jnp.int32, L) < VOCAB).astype(jnp.float32)

    def zero_buf(buf):
        def body(i, _):
            r = i // (HIDDEN // L)
            k = i - r * (HIDDEN // L)
            buf[r, pl.ds(k * L, L)] = zero
            return 0
        lax.fori_loop(0, C * (HIDDEN // L), body, 0, unroll=8)

    zero_buf(buf0)
    zero_buf(buf1)

    pltpu.async_copy(in_hbm.at[pl.ds(wid * IN_STRIP, IN_STRIP)],
                     in_v.at[pl.ds(0, IN_STRIP)], si).wait()

    def chunk(ci, buf, sem_out):
        def body(r, _):
            vals = in_v[pl.ds((ci * C + r) * VOCAB, L)]
            buf[r, pl.ds(0, L)] = vals * mask6
            return 0
        lax.fori_loop(0, C, body, 0, unroll=4)
        rb = rb0 + ci * C
        return pltpu.async_copy(buf, out_hbm.at[b, pl.ds(rb, C), :], sem_out)

    dma0 = chunk(0, buf0, so0)
    dma1 = chunk(1, buf1, so1)
    for ci in range(2, NCHUNK, 2):
        dma0.wait()
        dma0 = chunk(ci, buf0, so0)
        dma1.wait()
        dma1 = chunk(ci + 1, buf1, so1)
    dma0.wait()
    dma1.wait()


def kernel(input_probs):
    flat_in = input_probs.reshape(ROWS * VOCAB)
    return pl.kernel(
        _sc_pad_body,
        out_type=jax.ShapeDtypeStruct((BATCH, SEQ, HIDDEN), jnp.float32),
        mesh=plsc.VectorSubcoreMesh(core_axis_name="c", subcore_axis_name="s"),
        scratch_types=[
            pltpu.VMEM((IN_STRIP + L,), jnp.float32),
            pltpu.VMEM((C, HIDDEN), jnp.float32),
            pltpu.VMEM((C, HIDDEN), jnp.float32),
            pltpu.SemaphoreType.DMA,
            pltpu.SemaphoreType.DMA,
            pltpu.SemaphoreType.DMA,
        ],
        compiler_params=pltpu.CompilerParams(
            use_tc_tiling_on_sc=True, needs_layout_passes=False),
    )(flat_in)


# speedup vs baseline: 1.6799x; 1.6799x over previous
import jax
import jax.numpy as jnp
from jax.experimental import pallas as pl

VOCAB = 6
HIDDEN = 768
ROWS = 4 * 8192
BLK = 4096


def _pad_kernel(in_ref, out_ref):
    i = pl.program_id(0)
    x = in_ref[...]
    first = jnp.concatenate(
        [x, jnp.zeros((BLK, 128 - VOCAB), x.dtype)], axis=-1)
    out_ref[:, 0:128] = first

    @pl.when(i < 2)
    def _():
        out_ref[:, 128:] = jnp.zeros((BLK, HIDDEN - 128), x.dtype)


def kernel(input_probs):
    flat = input_probs.reshape(ROWS, VOCAB)
    out = pl.pallas_call(
        _pad_kernel,
        grid=(ROWS // BLK,),
        in_specs=[pl.BlockSpec((BLK, VOCAB), lambda i: (i, 0))],
        out_specs=pl.BlockSpec((BLK, HIDDEN), lambda i: (i, 0)),
        out_shape=jax.ShapeDtypeStruct((ROWS, HIDDEN), input_probs.dtype),
    )(flat)
    return out.reshape(input_probs.shape[0], input_probs.shape[1], HIDDEN)


# speedup vs baseline: 1.6916x; 1.0070x over previous
import jax
import jax.numpy as jnp
from jax.experimental import pallas as pl

VOCAB = 6
HIDDEN = 768
ROWS = 4 * 8192
BLK = 8192


def _pad_kernel(in_ref, out_ref):
    i = pl.program_id(0)
    x = in_ref[...]
    first = jnp.concatenate(
        [x, jnp.zeros((BLK, 128 - VOCAB), x.dtype)], axis=-1)
    out_ref[:, 0:128] = first

    @pl.when(i < 2)
    def _():
        out_ref[:, 128:] = jnp.zeros((BLK, HIDDEN - 128), x.dtype)


def kernel(input_probs):
    flat = input_probs.reshape(ROWS, VOCAB)
    out = pl.pallas_call(
        _pad_kernel,
        grid=(ROWS // BLK,),
        in_specs=[pl.BlockSpec((BLK, VOCAB), lambda i: (i, 0))],
        out_specs=pl.BlockSpec((BLK, HIDDEN), lambda i: (i, 0)),
        out_shape=jax.ShapeDtypeStruct((ROWS, HIDDEN), input_probs.dtype),
    )(flat)
    return out.reshape(input_probs.shape[0], input_probs.shape[1], HIDDEN)
